# baseline (speedup 1.0000x reference)
import functools

import numpy as np
import jax
import jax.numpy as jnp
from jax.experimental import pallas as pl
from jax.experimental.pallas import tpu as pltpu

_EPS = 1e-6
_NBUF = 8
_TILE_N = 1


_LOG2E = 1.4426950408889634


def _gem_body(p_ref, x_hbm, st_ref, o_ref, x_buf, in_sem):
    p = p_ref[0]
    p_l2e = p * _LOG2E
    inv_p_l2e = _LOG2E / p
    n_chunks = x_hbm.shape[0] // _TILE_N

    def start_in(slot, chunk):
        pltpu.make_async_copy(
            x_hbm.at[pl.ds(chunk * _TILE_N, _TILE_N)],
            x_buf.at[slot], in_sem.at[slot]).start()

    def wait_in(slot):
        pltpu.make_async_copy(
            x_hbm.at[pl.ds(0, _TILE_N)],
            x_buf.at[slot], in_sem.at[slot]).wait()

    for c0 in range(min(_NBUF, n_chunks)):
        start_in(c0, c0)

    def body(i, _):
        s = jax.lax.rem(i, _NBUF)
        wait_in(s)
        for b in range(_TILE_N):
            xc = jnp.maximum(x_buf[s, b], _EPS)
            y = (jax.lax.log(xc) * p_l2e).astype(jnp.bfloat16)
            zp = jnp.exp2(y)
            pooled = jnp.dot(st_ref[...], zp,
                             preferred_element_type=jnp.float32)
            o_ref[i * _TILE_N + b] = jnp.exp2(jax.lax.log(pooled) * inv_p_l2e)

        @pl.when(i + _NBUF < n_chunks)
        def _():
            start_in(s, i + _NBUF)
        return 0

    jax.lax.fori_loop(0, n_chunks, body, 0)


def _segment_mean_matrix_t(hw, bins):
    seg = hw // bins
    m = np.zeros((bins, hw), dtype=np.float32)
    m[np.arange(hw) // seg, np.arange(hw)] = 1.0 / seg
    return jnp.asarray(m, dtype=jnp.bfloat16)


@functools.partial(jax.jit, static_argnames=("bins",))
def _gem_hpp_nhwc(xt, p_scalar, bins):
    n, hw, c = xt.shape
    st = _segment_mean_matrix_t(hw, bins)

    return pl.pallas_call(
        _gem_body,
        out_shape=jax.ShapeDtypeStruct((n, bins, c), xt.dtype),
        in_specs=[
            pl.BlockSpec(memory_space=pltpu.MemorySpace.SMEM),
            pl.BlockSpec(memory_space=pltpu.MemorySpace.HBM),
            pl.BlockSpec(memory_space=pltpu.MemorySpace.VMEM),
        ],
        out_specs=pl.BlockSpec(memory_space=pltpu.MemorySpace.VMEM),
        scratch_shapes=[
            pltpu.VMEM((_NBUF, _TILE_N, hw, c), jnp.float32),
            pltpu.SemaphoreType.DMA((_NBUF,)),
        ],
    )(p_scalar, xt, st)


def kernel(x, p_scalar):
    n, c, h, w = x.shape
    bins = 64
    xt = x.transpose(0, 2, 3, 1).reshape(n, h * w, c)
    out = _gem_hpp_nhwc(xt, p_scalar, bins)
    return out.transpose(0, 2, 1)


# speedup vs baseline: 1.0451x; 1.0451x over previous
import functools

import numpy as np
import jax
import jax.numpy as jnp
from jax.experimental import pallas as pl
from jax.experimental.pallas import tpu as pltpu

_EPS = 1e-6
_NBUF = 4
_TILE_N = 4


_LOG2E = 1.4426950408889634


def _gem_body(p_ref, x_hbm, st_ref, o_ref, x_buf, in_sem):
    p = p_ref[0]
    p_l2e = p * _LOG2E
    inv_p_l2e = _LOG2E / p
    n_chunks = x_hbm.shape[0] // _TILE_N

    def start_in(slot, chunk):
        pltpu.make_async_copy(
            x_hbm.at[pl.ds(chunk * _TILE_N, _TILE_N)],
            x_buf.at[slot], in_sem.at[slot]).start()

    def wait_in(slot):
        pltpu.make_async_copy(
            x_hbm.at[pl.ds(0, _TILE_N)],
            x_buf.at[slot], in_sem.at[slot]).wait()

    for c0 in range(min(_NBUF, n_chunks)):
        start_in(c0, c0)

    def body(i, _):
        s = jax.lax.rem(i, _NBUF)
        wait_in(s)
        for b in range(_TILE_N):
            xc = jnp.maximum(x_buf[s, b], _EPS)
            y = (jax.lax.log(xc) * p_l2e).astype(jnp.bfloat16)
            zp = jnp.exp2(y)
            pooled = jnp.dot(st_ref[...], zp,
                             preferred_element_type=jnp.float32)
            o_ref[i * _TILE_N + b] = jnp.exp2(jax.lax.log(pooled) * inv_p_l2e)

        @pl.when(i + _NBUF < n_chunks)
        def _():
            start_in(s, i + _NBUF)
        return 0

    jax.lax.fori_loop(0, n_chunks, body, 0)


def _segment_mean_matrix_t(hw, bins):
    seg = hw // bins
    m = np.zeros((bins, hw), dtype=np.float32)
    m[np.arange(hw) // seg, np.arange(hw)] = 1.0 / seg
    return jnp.asarray(m, dtype=jnp.bfloat16)


@functools.partial(jax.jit, static_argnames=("bins",))
def _gem_hpp_nhwc(xt, p_scalar, bins):
    n, hw, c = xt.shape
    st = _segment_mean_matrix_t(hw, bins)

    return pl.pallas_call(
        _gem_body,
        out_shape=jax.ShapeDtypeStruct((n, bins, c), xt.dtype),
        in_specs=[
            pl.BlockSpec(memory_space=pltpu.MemorySpace.SMEM),
            pl.BlockSpec(memory_space=pltpu.MemorySpace.HBM),
            pl.BlockSpec(memory_space=pltpu.MemorySpace.VMEM),
        ],
        out_specs=pl.BlockSpec(memory_space=pltpu.MemorySpace.VMEM),
        scratch_shapes=[
            pltpu.VMEM((_NBUF, _TILE_N, hw, c), jnp.float32),
            pltpu.SemaphoreType.DMA((_NBUF,)),
        ],
    )(p_scalar, xt, st)


def kernel(x, p_scalar):
    n, c, h, w = x.shape
    bins = 64
    xt = x.transpose(0, 2, 3, 1).reshape(n, h * w, c)
    out = _gem_hpp_nhwc(xt, p_scalar, bins)
    return out.transpose(0, 2, 1)


# speedup vs baseline: 1.0956x; 1.0483x over previous
import functools

import numpy as np
import jax
import jax.numpy as jnp
from jax.experimental import pallas as pl
from jax.experimental.pallas import tpu as pltpu

_EPS = 1e-6
_NBUF = 4
_TILE_N = 2
_LOG2E = 1.4426950408889634


def _gem_body(p_ref, x_hbm, st_ref, o_hbm, x_buf, o_buf, in_sem, out_sem):
    p = p_ref[0]
    p_l2e = p * _LOG2E
    inv_p_l2e = _LOG2E / p
    n_chunks = x_hbm.shape[0] // _TILE_N

    def start_in(slot, chunk):
        pltpu.make_async_copy(
            x_hbm.at[pl.ds(chunk * _TILE_N, _TILE_N)],
            x_buf.at[slot], in_sem.at[slot]).start()

    def wait_in(slot):
        pltpu.make_async_copy(
            x_hbm.at[pl.ds(0, _TILE_N)],
            x_buf.at[slot], in_sem.at[slot]).wait()

    def start_out(slot, chunk):
        pltpu.make_async_copy(
            o_buf.at[slot],
            o_hbm.at[pl.ds(chunk * _TILE_N, _TILE_N)],
            out_sem.at[slot]).start()

    def wait_out(slot):
        pltpu.make_async_copy(
            o_buf.at[slot],
            o_hbm.at[pl.ds(0, _TILE_N)],
            out_sem.at[slot]).wait()

    for c0 in range(min(_NBUF, n_chunks)):
        start_in(c0, c0)

    def body(i, _):
        s = jax.lax.rem(i, _NBUF)
        wait_in(s)

        @pl.when(i >= _NBUF)
        def _():
            wait_out(s)

        for b in range(_TILE_N):
            xc = jnp.maximum(x_buf[s, b], _EPS)
            y = (jax.lax.log(xc) * p_l2e).astype(jnp.bfloat16)
            zp = jnp.exp2(y)
            pooled = jnp.dot(st_ref[...], zp,
                             preferred_element_type=jnp.float32)
            o_buf[s, b] = jnp.exp2(jax.lax.log(pooled) * inv_p_l2e)
        start_out(s, i)

        @pl.when(i + _NBUF < n_chunks)
        def _():
            start_in(s, i + _NBUF)
        return 0

    jax.lax.fori_loop(0, n_chunks, body, 0)

    for c0 in range(min(_NBUF, n_chunks)):
        wait_out(jax.lax.rem(jnp.int32(max(n_chunks - _NBUF, 0) + c0), _NBUF))


def _segment_mean_matrix_t(hw, bins):
    seg = hw // bins
    m = np.zeros((bins, hw), dtype=np.float32)
    m[np.arange(hw) // seg, np.arange(hw)] = 1.0 / seg
    return jnp.asarray(m, dtype=jnp.bfloat16)


@functools.partial(jax.jit, static_argnames=("bins",))
def _gem_hpp_nhwc(xt, p_scalar, bins):
    n, hw, c = xt.shape
    st = _segment_mean_matrix_t(hw, bins)

    return pl.pallas_call(
        _gem_body,
        out_shape=jax.ShapeDtypeStruct((n, bins, c), xt.dtype),
        in_specs=[
            pl.BlockSpec(memory_space=pltpu.MemorySpace.SMEM),
            pl.BlockSpec(memory_space=pltpu.MemorySpace.HBM),
            pl.BlockSpec(memory_space=pltpu.MemorySpace.VMEM),
        ],
        out_specs=pl.BlockSpec(memory_space=pltpu.MemorySpace.HBM),
        scratch_shapes=[
            pltpu.VMEM((_NBUF, _TILE_N, hw, c), jnp.float32),
            pltpu.VMEM((_NBUF, _TILE_N, bins, c), jnp.float32),
            pltpu.SemaphoreType.DMA((_NBUF,)),
            pltpu.SemaphoreType.DMA((_NBUF,)),
        ],
    )(p_scalar, xt, st)


def kernel(x, p_scalar):
    n, c, h, w = x.shape
    bins = 64
    xt = x.transpose(0, 2, 3, 1).reshape(n, h * w, c)
    out = _gem_hpp_nhwc(xt, p_scalar, bins)
    return out.transpose(0, 2, 1)


# speedup vs baseline: 1.1276x; 1.0292x over previous
import functools

import numpy as np
import jax
import jax.numpy as jnp
from jax.experimental import pallas as pl
from jax.experimental.pallas import tpu as pltpu

_EPS = 1e-6
_NBUF = 4
_TILE_N = 2
_LOG2E = 1.4426950408889634


def _gem_body(p_ref, x_hbm, st_ref, o_hbm, x_buf, o_buf, in_sem, out_sem):
    p = p_ref[0]
    p_l2e = p * _LOG2E
    inv_p_l2e = _LOG2E / p
    n_chunks = x_hbm.shape[0] // _TILE_N

    def start_in(slot, chunk):
        pltpu.make_async_copy(
            x_hbm.at[pl.ds(chunk * _TILE_N, _TILE_N)],
            x_buf.at[slot], in_sem.at[slot]).start()

    def wait_in(slot):
        pltpu.make_async_copy(
            x_hbm.at[pl.ds(0, _TILE_N)],
            x_buf.at[slot], in_sem.at[slot]).wait()

    def start_out(slot, chunk):
        pltpu.make_async_copy(
            o_buf.at[slot],
            o_hbm.at[pl.ds(chunk * _TILE_N, _TILE_N)],
            out_sem.at[slot]).start()

    def wait_out(slot):
        pltpu.make_async_copy(
            o_buf.at[slot],
            o_hbm.at[pl.ds(0, _TILE_N)],
            out_sem.at[slot]).wait()

    for c0 in range(min(_NBUF, n_chunks)):
        start_in(c0, c0)

    def body(i, _):
        s = jax.lax.rem(i, _NBUF)
        wait_in(s)

        @pl.when(i >= _NBUF)
        def _():
            wait_out(s)

        for b in range(_TILE_N):
            xc = jnp.maximum(x_buf[s, b], _EPS)
            zp = jnp.exp2(jax.lax.log(xc) * p_l2e)
            pooled = jnp.dot(st_ref[...], zp,
                             preferred_element_type=jnp.float32)
            o_buf[s, b] = jnp.exp2(jax.lax.log(pooled) * inv_p_l2e)
        start_out(s, i)

        @pl.when(i + _NBUF < n_chunks)
        def _():
            start_in(s, i + _NBUF)
        return 0

    jax.lax.fori_loop(0, n_chunks, body, 0)

    for c0 in range(min(_NBUF, n_chunks)):
        wait_out(jax.lax.rem(jnp.int32(max(n_chunks - _NBUF, 0) + c0), _NBUF))


def _segment_mean_matrix_t(hw, bins):
    seg = hw // bins
    m = np.zeros((bins, hw), dtype=np.float32)
    m[np.arange(hw) // seg, np.arange(hw)] = 1.0 / seg
    return jnp.asarray(m)


@functools.partial(jax.jit, static_argnames=("bins",))
def _gem_hpp_nhwc(xt, p_scalar, bins):
    n, hw, c = xt.shape
    st = _segment_mean_matrix_t(hw, bins)

    return pl.pallas_call(
        _gem_body,
        out_shape=jax.ShapeDtypeStruct((n, bins, c), xt.dtype),
        in_specs=[
            pl.BlockSpec(memory_space=pltpu.MemorySpace.SMEM),
            pl.BlockSpec(memory_space=pltpu.MemorySpace.HBM),
            pl.BlockSpec(memory_space=pltpu.MemorySpace.VMEM),
        ],
        out_specs=pl.BlockSpec(memory_space=pltpu.MemorySpace.HBM),
        scratch_shapes=[
            pltpu.VMEM((_NBUF, _TILE_N, hw, c), jnp.float32),
            pltpu.VMEM((_NBUF, _TILE_N, bins, c), jnp.float32),
            pltpu.SemaphoreType.DMA((_NBUF,)),
            pltpu.SemaphoreType.DMA((_NBUF,)),
        ],
    )(p_scalar, xt, st)


def kernel(x, p_scalar):
    n, c, h, w = x.shape
    bins = 64
    xt = x.transpose(0, 2, 3, 1).reshape(n, h * w, c)
    out = _gem_hpp_nhwc(xt, p_scalar, bins)
    return out.transpose(0, 2, 1)
